# baseline (speedup 1.0000x reference)
import functools

import jax
import jax.numpy as jnp
import numpy as np
from jax import lax
from jax.experimental import pallas as pl
from jax.experimental.pallas import tpu as pltpu
from jax.experimental.pallas import tpu_sc as plsc

_VOCAB = 100000
_EMBED_DIM = 64
_NUM_SAMPLED = 100
_BATCH = 4096
_SPAD = 128

_NC = 2
_NS = 16
_NW = _NC * _NS
_BPW = _BATCH // _NW


@functools.cache
def _make_sc_gather():
    @functools.partial(
        pl.kernel,
        out_type=[
            jax.ShapeDtypeStruct((_BATCH, _EMBED_DIM), jnp.float32),
            jax.ShapeDtypeStruct((_SPAD, _EMBED_DIM), jnp.float32),
        ],
        mesh=plsc.VectorSubcoreMesh(
            core_axis_name="c", subcore_axis_name="s",
            num_cores=_NC, num_subcores=_NS,
        ),
        scratch_types=[
            pltpu.VMEM((_BPW,), jnp.int32),
            pltpu.VMEM((_BPW, _EMBED_DIM), jnp.float32),
            pltpu.SemaphoreType.DMA,
            pltpu.VMEM((_SPAD,), jnp.int32),
            pltpu.VMEM((_SPAD, _EMBED_DIM), jnp.float32),
        ],
        compiler_params=pltpu.CompilerParams(use_tc_tiling_on_sc=False),
    )
    def _sc_gather(table_hbm, tid_hbm, sid_hbm, true_out, samp_out,
                   idx_v, rows_v, sem, sidx_v, srows_v):
        wid = lax.axis_index("s") * _NC + lax.axis_index("c")
        base = wid * _BPW
        pltpu.sync_copy(tid_hbm.at[pl.ds(base, _BPW)], idx_v)
        pltpu.async_copy(table_hbm.at[idx_v], rows_v, sem).wait()
        pltpu.sync_copy(rows_v, true_out.at[pl.ds(base, _BPW)])

        @pl.when(wid == 0)
        def _():
            pltpu.sync_copy(sid_hbm, sidx_v)
            pltpu.async_copy(table_hbm.at[sidx_v], srows_v, sem).wait()
            pltpu.sync_copy(srows_v, samp_out)

    return _sc_gather


def _expected_count(ids_f32):
    p = (jnp.log(ids_f32 + 2.0) - jnp.log(ids_f32 + 1.0)) / np.log(_VOCAB + 1.0)
    l1p = -p * (1.0 + p * (1 / 2 + p * (1 / 3 + p * (1 / 4 + p * (1 / 5 + p / 6)))))
    t = float(_NUM_SAMPLED) * l1p
    em_small = t * (1.0 + t * (1 / 2 + t * (1 / 6 + t * (1 / 24 + t / 120))))
    em = jnp.where(t > -0.125, em_small, jnp.exp(t) - 1.0)
    return -em


def _tc_body(x_ref, tw_ref, sw_ref, tid_ref, sid_ref, out_ref):
    x = x_ref[...]
    tw = tw_ref[...]
    sw = sw_ref[...]
    tid = tid_ref[...]
    sid = sid_ref[...]

    true_logits = jnp.sum(x * tw, axis=1, keepdims=True)
    sampled_logits = lax.dot_general(
        x, sw, (((1,), (1,)), ((), ())),
        preferred_element_type=jnp.float32)

    tl = true_logits - jnp.log(_expected_count(tid.astype(jnp.float32)))
    sl = sampled_logits - jnp.log(_expected_count(sid.astype(jnp.float32)))

    sl = jnp.where(sid == tid, sl - 1e9, sl)
    col = lax.broadcasted_iota(jnp.int32, sl.shape, 1)
    sl = jnp.where(col < _NUM_SAMPLED, sl, -1e30)

    m = jnp.maximum(jnp.max(sl, axis=1, keepdims=True), tl)
    s = jnp.sum(jnp.exp(sl - m), axis=1, keepdims=True) + jnp.exp(tl - m)
    out_ref[...] = jnp.log(s) + m - tl


def _tc_dense(x, tw, sw, tid, sid):
    nblk = 8
    bb = _BATCH // nblk
    return pl.pallas_call(
        _tc_body,
        grid=(nblk,),
        in_specs=[
            pl.BlockSpec((bb, _EMBED_DIM), lambda i: (i, 0)),
            pl.BlockSpec((bb, _EMBED_DIM), lambda i: (i, 0)),
            pl.BlockSpec((_SPAD, _EMBED_DIM), lambda i: (0, 0)),
            pl.BlockSpec((bb, 1), lambda i: (i, 0)),
            pl.BlockSpec((1, _SPAD), lambda i: (0, 0)),
        ],
        out_specs=pl.BlockSpec((bb, 1), lambda i: (i, 0)),
        out_shape=jax.ShapeDtypeStruct((_BATCH, 1), jnp.float32),
    )(x, tw, sw, tid, sid)


def kernel(inputs, label_idx, embeddings, zero_bias, sampled_ids):
    del zero_bias
    tid = label_idx.reshape(-1).astype(jnp.int32)
    sid_pad = jnp.concatenate(
        [sampled_ids.astype(jnp.int32),
         jnp.zeros((_SPAD - _NUM_SAMPLED,), jnp.int32)])
    true_w, samp_w = _make_sc_gather()(embeddings, tid, sid_pad)
    loss = _tc_dense(inputs, true_w, samp_w,
                     tid.reshape(_BATCH, 1), sid_pad.reshape(1, _SPAD))
    return loss


# speedup vs baseline: 1.4089x; 1.4089x over previous
import functools

import jax
import jax.numpy as jnp
import numpy as np
from jax import lax
from jax.experimental import pallas as pl
from jax.experimental.pallas import tpu as pltpu
from jax.experimental.pallas import tpu_sc as plsc

_VOCAB = 100000
_EMBED_DIM = 64
_NUM_SAMPLED = 100
_BATCH = 4096
_SPAD = 128

_NC = 2
_NS = 16
_NW = _NC * _NS
_BPW = _BATCH // _NW


@functools.cache
def _make_sc_gather():
    @functools.partial(
        pl.kernel,
        out_type=[
            jax.ShapeDtypeStruct((_BATCH, _EMBED_DIM), jnp.float32),
            jax.ShapeDtypeStruct((_SPAD, _EMBED_DIM), jnp.float32),
        ],
        mesh=plsc.VectorSubcoreMesh(
            core_axis_name="c", subcore_axis_name="s",
            num_cores=_NC, num_subcores=_NS,
        ),
        scratch_types=[
            pltpu.VMEM((_BPW,), jnp.int32),
            pltpu.VMEM((_BPW, _EMBED_DIM), jnp.float32),
            pltpu.SemaphoreType.DMA,
            pltpu.VMEM((16,), jnp.int32),
            pltpu.VMEM((16, _EMBED_DIM), jnp.float32),
            pltpu.SemaphoreType.DMA,
        ],
        compiler_params=pltpu.CompilerParams(needs_layout_passes=False),
    )
    def _sc_gather(table_hbm, tid_hbm, sid_hbm, true_out, samp_out,
                   idx_v, rows_v, sem, sidx_v, srows_v, ssem):
        wid = lax.axis_index("s") * _NC + lax.axis_index("c")
        base = wid * _BPW
        lanes = lax.iota(jnp.int32, 16)
        pltpu.sync_copy(tid_hbm.at[pl.ds(base, _BPW)], idx_v)

        descs = []
        for c in range(_BPW // 16):
            vec = idx_v[pl.ds(c * 16, 16)]
            for l in range(16):
                row = jnp.sum(jnp.where(lanes == l, vec, 0))
                descs.append(pltpu.async_copy(
                    table_hbm.at[pl.ds(row, 1)],
                    rows_v.at[pl.ds(c * 16 + l, 1)], sem))

        @pl.when(wid < _SPAD // 16)
        def _():
            pltpu.sync_copy(sid_hbm.at[pl.ds(wid * 16, 16)], sidx_v)
            svec = sidx_v[...]
            sdescs = []
            for l in range(16):
                row = jnp.sum(jnp.where(lanes == l, svec, 0))
                sdescs.append(pltpu.async_copy(
                    table_hbm.at[pl.ds(row, 1)],
                    srows_v.at[pl.ds(l, 1)], ssem))
            for d in sdescs:
                d.wait()
            pltpu.sync_copy(srows_v, samp_out.at[pl.ds(wid * 16, 16)])

        for d in descs:
            d.wait()
        pltpu.sync_copy(rows_v, true_out.at[pl.ds(base, _BPW)])

    return _sc_gather


def _expected_count(ids_f32):
    p = (jnp.log(ids_f32 + 2.0) - jnp.log(ids_f32 + 1.0)) / np.log(_VOCAB + 1.0)
    l1p = -p * (1.0 + p * (1 / 2 + p * (1 / 3 + p * (1 / 4 + p * (1 / 5 + p / 6)))))
    t = float(_NUM_SAMPLED) * l1p
    em_small = t * (1.0 + t * (1 / 2 + t * (1 / 6 + t * (1 / 24 + t / 120))))
    em = jnp.where(t > -0.125, em_small, jnp.exp(t) - 1.0)
    return -em


def _tc_body(x_ref, tw_ref, sw_ref, tid_ref, sid_ref, out_ref):
    x = x_ref[...]
    tw = tw_ref[...]
    sw = sw_ref[...]
    tid = tid_ref[...]
    sid = sid_ref[...]

    true_logits = jnp.sum(x * tw, axis=1, keepdims=True)
    sampled_logits = lax.dot_general(
        x, sw, (((1,), (1,)), ((), ())),
        preferred_element_type=jnp.float32)

    tl = true_logits - jnp.log(_expected_count(tid.astype(jnp.float32)))
    sl = sampled_logits - jnp.log(_expected_count(sid.astype(jnp.float32)))

    sl = jnp.where(sid == tid, sl - 1e9, sl)
    col = lax.broadcasted_iota(jnp.int32, sl.shape, 1)
    sl = jnp.where(col < _NUM_SAMPLED, sl, -1e30)

    m = jnp.maximum(jnp.max(sl, axis=1, keepdims=True), tl)
    s = jnp.sum(jnp.exp(sl - m), axis=1, keepdims=True) + jnp.exp(tl - m)
    out_ref[...] = jnp.log(s) + m - tl


def _tc_dense(x, tw, sw, tid, sid):
    nblk = 8
    bb = _BATCH // nblk
    return pl.pallas_call(
        _tc_body,
        grid=(nblk,),
        in_specs=[
            pl.BlockSpec((bb, _EMBED_DIM), lambda i: (i, 0)),
            pl.BlockSpec((bb, _EMBED_DIM), lambda i: (i, 0)),
            pl.BlockSpec((_SPAD, _EMBED_DIM), lambda i: (0, 0)),
            pl.BlockSpec((bb, 1), lambda i: (i, 0)),
            pl.BlockSpec((1, _SPAD), lambda i: (0, 0)),
        ],
        out_specs=pl.BlockSpec((bb, 1), lambda i: (i, 0)),
        out_shape=jax.ShapeDtypeStruct((_BATCH, 1), jnp.float32),
    )(x, tw, sw, tid, sid)


def kernel(inputs, label_idx, embeddings, zero_bias, sampled_ids):
    del zero_bias
    tid = label_idx.reshape(-1).astype(jnp.int32)
    sid_pad = jnp.concatenate(
        [sampled_ids.astype(jnp.int32),
         jnp.zeros((_SPAD - _NUM_SAMPLED,), jnp.int32)])
    true_w, samp_w = _make_sc_gather()(embeddings, tid, sid_pad)
    loss = _tc_dense(inputs, true_w, samp_w,
                     tid.reshape(_BATCH, 1), sid_pad.reshape(1, _SPAD))
    return loss


# speedup vs baseline: 1.4359x; 1.0192x over previous
import functools

import jax
import jax.numpy as jnp
import numpy as np
from jax import lax
from jax.experimental import pallas as pl
from jax.experimental.pallas import tpu as pltpu
from jax.experimental.pallas import tpu_sc as plsc

_VOCAB = 100000
_EMBED_DIM = 64
_NUM_SAMPLED = 100
_BATCH = 4096
_SPAD = 128
_ROWPAD = 128

_NC = 2
_NS = 16
_NW = _NC * _NS
_BPW = _BATCH // _NW

_CB = 4096


def _repack_body(ett_ref, o_ref):
    o_ref[:, : _EMBED_DIM] = ett_ref[...].T


def _repack(ett):
    nblk = (_VOCAB + _CB - 1) // _CB
    return pl.pallas_call(
        _repack_body,
        grid=(nblk,),
        in_specs=[pl.BlockSpec((_EMBED_DIM, _CB), lambda i: (0, i))],
        out_specs=pl.BlockSpec((_CB, _ROWPAD), lambda i: (i, 0)),
        out_shape=jax.ShapeDtypeStruct((_VOCAB, _ROWPAD), jnp.float32),
    )(ett)


@functools.cache
def _make_sc_gather():
    @functools.partial(
        pl.kernel,
        out_type=[
            jax.ShapeDtypeStruct((_BATCH, _ROWPAD), jnp.float32),
            jax.ShapeDtypeStruct((_SPAD, _ROWPAD), jnp.float32),
        ],
        mesh=plsc.VectorSubcoreMesh(
            core_axis_name="c", subcore_axis_name="s",
            num_cores=_NC, num_subcores=_NS,
        ),
        scratch_types=[
            pltpu.VMEM((_BPW,), jnp.int32),
            pltpu.VMEM((_BPW, _ROWPAD), jnp.float32),
            pltpu.SemaphoreType.DMA,
            pltpu.VMEM((16,), jnp.int32),
            pltpu.VMEM((16, _ROWPAD), jnp.float32),
            pltpu.SemaphoreType.DMA,
        ],
    )
    def _sc_gather(o_hbm, tid_hbm, sid_hbm, true_out, samp_out,
                   idx_v, rows_v, sem, sidx_v, srows_v, ssem):
        wid = lax.axis_index("s") * _NC + lax.axis_index("c")
        base = wid * _BPW
        pltpu.sync_copy(tid_hbm.at[pl.ds(base, _BPW)], idx_v)
        d = pltpu.async_copy(o_hbm.at[idx_v], rows_v, sem)

        @pl.when(wid < _SPAD // 16)
        def _():
            pltpu.sync_copy(sid_hbm.at[pl.ds(wid * 16, 16)], sidx_v)
            pltpu.async_copy(o_hbm.at[sidx_v], srows_v, ssem).wait()
            pltpu.sync_copy(srows_v, samp_out.at[pl.ds(wid * 16, 16)])

        d.wait()
        pltpu.sync_copy(rows_v, true_out.at[pl.ds(base, _BPW)])

    return _sc_gather


def _expected_count(ids_f32):
    p = (jnp.log(ids_f32 + 2.0) - jnp.log(ids_f32 + 1.0)) / np.log(_VOCAB + 1.0)
    l1p = -p * (1.0 + p * (1 / 2 + p * (1 / 3 + p * (1 / 4 + p * (1 / 5 + p / 6)))))
    t = float(_NUM_SAMPLED) * l1p
    em_small = t * (1.0 + t * (1 / 2 + t * (1 / 6 + t * (1 / 24 + t / 120))))
    em = jnp.where(t > -0.125, em_small, jnp.exp(t) - 1.0)
    return -em


def _tc_body(x_ref, tw_ref, sw_ref, tid_ref, sid_ref, out_ref):
    x = x_ref[...]
    tw = tw_ref[:, : _EMBED_DIM]
    sw = sw_ref[:, : _EMBED_DIM]
    tid = tid_ref[...]
    sid = sid_ref[...]

    true_logits = jnp.sum(x * tw, axis=1, keepdims=True)
    sampled_logits = lax.dot_general(
        x, sw, (((1,), (1,)), ((), ())),
        preferred_element_type=jnp.float32)

    tl = true_logits - jnp.log(_expected_count(tid.astype(jnp.float32)))
    sl = sampled_logits - jnp.log(_expected_count(sid.astype(jnp.float32)))

    sl = jnp.where(sid == tid, sl - 1e9, sl)
    col = lax.broadcasted_iota(jnp.int32, sl.shape, 1)
    sl = jnp.where(col < _NUM_SAMPLED, sl, -1e30)

    m = jnp.maximum(jnp.max(sl, axis=1, keepdims=True), tl)
    s = jnp.sum(jnp.exp(sl - m), axis=1, keepdims=True) + jnp.exp(tl - m)
    out_ref[...] = jnp.log(s) + m - tl


def _tc_dense(x, trows, srows, tid, sid):
    nblk = 8
    bb = _BATCH // nblk
    return pl.pallas_call(
        _tc_body,
        grid=(nblk,),
        in_specs=[
            pl.BlockSpec((bb, _EMBED_DIM), lambda i: (i, 0)),
            pl.BlockSpec((bb, _ROWPAD), lambda i: (i, 0)),
            pl.BlockSpec((_SPAD, _ROWPAD), lambda i: (0, 0)),
            pl.BlockSpec((bb, 1), lambda i: (i, 0)),
            pl.BlockSpec((1, _SPAD), lambda i: (0, 0)),
        ],
        out_specs=pl.BlockSpec((bb, 1), lambda i: (i, 0)),
        out_shape=jax.ShapeDtypeStruct((_BATCH, 1), jnp.float32),
    )(x, trows, srows, tid, sid)


def kernel(inputs, label_idx, embeddings, zero_bias, sampled_ids):
    del zero_bias
    tid = label_idx.reshape(-1).astype(jnp.int32)
    sid_pad = jnp.concatenate(
        [sampled_ids.astype(jnp.int32),
         jnp.zeros((_SPAD - _NUM_SAMPLED,), jnp.int32)])
    staged = _repack(embeddings.T)
    true_rows, samp_rows = _make_sc_gather()(staged, tid, sid_pad)
    loss = _tc_dense(inputs, true_rows, samp_rows,
                     tid.reshape(_BATCH, 1), sid_pad.reshape(1, _SPAD))
    return loss


# speedup vs baseline: 2.1111x; 1.4702x over previous
import functools

import jax
import jax.numpy as jnp
import numpy as np
from jax import lax
from jax.experimental import pallas as pl
from jax.experimental.pallas import tpu as pltpu
from jax.experimental.pallas import tpu_sc as plsc

_VOCAB = 100000
_EMBED_DIM = 64
_NUM_SAMPLED = 100
_BATCH = 4096
_SPAD = 128

_NC = 2
_NS = 16
_NW = _NC * _NS
_DPW = _EMBED_DIM // _NW


@functools.cache
def _make_sc_gather():
    @functools.partial(
        pl.kernel,
        out_type=[
            jax.ShapeDtypeStruct((_EMBED_DIM, _BATCH), jnp.float32),
            jax.ShapeDtypeStruct((_EMBED_DIM, _SPAD), jnp.float32),
        ],
        mesh=plsc.VectorSubcoreMesh(
            core_axis_name="c", subcore_axis_name="s",
            num_cores=_NC, num_subcores=_NS,
        ),
        scratch_types=[
            pltpu.VMEM((1, _VOCAB), jnp.float32),
            pltpu.VMEM((_BATCH,), jnp.int32),
            pltpu.VMEM((1, _BATCH), jnp.float32),
            pltpu.VMEM((_SPAD,), jnp.int32),
            pltpu.VMEM((1, _SPAD), jnp.float32),
        ],
        compiler_params=pltpu.CompilerParams(needs_layout_passes=False),
    )
    def _sc_gather(ett_hbm, tid_hbm, sid_hbm, g_out, samp_out,
                   row_v, tid_v, g_v, sid_v, sg_v):
        wid = lax.axis_index("s") * _NC + lax.axis_index("c")
        pltpu.sync_copy(tid_hbm, tid_v)
        pltpu.sync_copy(sid_hbm, sid_v)
        for dd in range(_DPW):
            d = wid + dd * _NW
            pltpu.sync_copy(ett_hbm.at[pl.ds(d, 1)], row_v)
            row = row_v.at[0]
            for c in range(_BATCH // 16):
                idx = tid_v[pl.ds(c * 16, 16)]
                g_v[0, pl.ds(c * 16, 16)] = plsc.load_gather(row, [idx])
            pltpu.sync_copy(g_v, g_out.at[pl.ds(d, 1)])
            for c in range(_SPAD // 16):
                idx = sid_v[pl.ds(c * 16, 16)]
                sg_v[0, pl.ds(c * 16, 16)] = plsc.load_gather(row, [idx])
            pltpu.sync_copy(sg_v, samp_out.at[pl.ds(d, 1)])

    return _sc_gather


def _expected_count(ids_f32):
    p = (jnp.log(ids_f32 + 2.0) - jnp.log(ids_f32 + 1.0)) / np.log(_VOCAB + 1.0)
    l1p = -p * (1.0 + p * (1 / 2 + p * (1 / 3 + p * (1 / 4 + p * (1 / 5 + p / 6)))))
    t = float(_NUM_SAMPLED) * l1p
    em_small = t * (1.0 + t * (1 / 2 + t * (1 / 6 + t * (1 / 24 + t / 120))))
    em = jnp.where(t > -0.125, em_small, jnp.exp(t) - 1.0)
    return -em


def _tc_body(xt_ref, g_ref, st_ref, tid_ref, sid_ref, out_ref):
    xt = xt_ref[...]
    g = g_ref[...]
    st = st_ref[...]
    tid = tid_ref[...]
    sid = sid_ref[...]

    tl = jnp.sum(xt * g, axis=0, keepdims=True)
    slt = lax.dot_general(
        st, xt, (((0,), (0,)), ((), ())),
        preferred_element_type=jnp.float32)

    tl = tl - jnp.log(_expected_count(tid.astype(jnp.float32)))
    slt = slt - jnp.log(_expected_count(sid.astype(jnp.float32)))

    slt = jnp.where(sid == tid, slt - 1e9, slt)
    srow = lax.broadcasted_iota(jnp.int32, slt.shape, 0)
    slt = jnp.where(srow < _NUM_SAMPLED, slt, -1e30)

    m = jnp.maximum(jnp.max(slt, axis=0, keepdims=True), tl)
    s = jnp.sum(jnp.exp(slt - m), axis=0, keepdims=True) + jnp.exp(tl - m)
    out_ref[...] = (jnp.log(s) + m - tl).T


def _tc_dense(xt, g, st, tid, sid):
    nblk = 8
    bb = _BATCH // nblk
    return pl.pallas_call(
        _tc_body,
        grid=(nblk,),
        in_specs=[
            pl.BlockSpec((_EMBED_DIM, bb), lambda i: (0, i)),
            pl.BlockSpec((_EMBED_DIM, bb), lambda i: (0, i)),
            pl.BlockSpec((_EMBED_DIM, _SPAD), lambda i: (0, 0)),
            pl.BlockSpec((1, bb), lambda i: (0, i)),
            pl.BlockSpec((_SPAD, 1), lambda i: (0, 0)),
        ],
        out_specs=pl.BlockSpec((bb, 1), lambda i: (i, 0)),
        out_shape=jax.ShapeDtypeStruct((_BATCH, 1), jnp.float32),
    )(xt, g, st, tid, sid)


def kernel(inputs, label_idx, embeddings, zero_bias, sampled_ids):
    del zero_bias
    tid = label_idx.reshape(-1).astype(jnp.int32)
    sid_pad = jnp.concatenate(
        [sampled_ids.astype(jnp.int32),
         jnp.zeros((_SPAD - _NUM_SAMPLED,), jnp.int32)])
    g, samp_t = _make_sc_gather()(embeddings.T, tid, sid_pad)
    loss = _tc_dense(inputs.T, g, samp_t,
                     tid.reshape(1, _BATCH), sid_pad.reshape(_SPAD, 1))
    return loss


# speedup vs baseline: 2.2296x; 1.0561x over previous
import functools

import jax
import jax.numpy as jnp
import numpy as np
from jax import lax
from jax.experimental import pallas as pl
from jax.experimental.pallas import tpu as pltpu
from jax.experimental.pallas import tpu_sc as plsc

_VOCAB = 100000
_EMBED_DIM = 64
_NUM_SAMPLED = 100
_BATCH = 4096
_SPAD = 128

_NC = 2
_NS = 16
_NW = _NC * _NS
_DPW = _EMBED_DIM // _NW


@functools.cache
def _make_sc_gather():
    @functools.partial(
        pl.kernel,
        out_type=[
            jax.ShapeDtypeStruct((_EMBED_DIM, _BATCH), jnp.float32),
            jax.ShapeDtypeStruct((_EMBED_DIM, _SPAD), jnp.float32),
        ],
        mesh=plsc.VectorSubcoreMesh(
            core_axis_name="c", subcore_axis_name="s",
            num_cores=_NC, num_subcores=_NS,
        ),
        scratch_types=[
            pltpu.VMEM((1, _VOCAB), jnp.float32),
            pltpu.VMEM((_BATCH,), jnp.int32),
            pltpu.VMEM((1, _BATCH), jnp.float32),
            pltpu.VMEM((_SPAD,), jnp.int32),
            pltpu.VMEM((1, _SPAD), jnp.float32),
        ],
        compiler_params=pltpu.CompilerParams(needs_layout_passes=False),
    )
    def _sc_gather(ett_hbm, tid_hbm, sid_hbm, g_out, samp_out,
                   row_v, tid_v, g_v, sid_v, sg_v):
        wid = lax.axis_index("s") * _NC + lax.axis_index("c")
        pltpu.sync_copy(tid_hbm, tid_v)
        pltpu.sync_copy(sid_hbm, sid_v)
        for dd in range(_DPW):
            d = wid + dd * _NW
            pltpu.sync_copy(ett_hbm.at[pl.ds(d, 1)], row_v)
            row = row_v.at[0]
            for c in range(_BATCH // 16):
                idx = tid_v[pl.ds(c * 16, 16)]
                g_v[0, pl.ds(c * 16, 16)] = plsc.load_gather(row, [idx])
            pltpu.sync_copy(g_v, g_out.at[pl.ds(d, 1)])
            for c in range(_SPAD // 16):
                idx = sid_v[pl.ds(c * 16, 16)]
                sg_v[0, pl.ds(c * 16, 16)] = plsc.load_gather(row, [idx])
            pltpu.sync_copy(sg_v, samp_out.at[pl.ds(d, 1)])

    return _sc_gather


def _expected_count(ids_f32):
    p = (jnp.log(ids_f32 + 2.0) - jnp.log(ids_f32 + 1.0)) / np.log(_VOCAB + 1.0)
    l1p = -p * (1.0 + p * (1 / 2 + p * (1 / 3 + p * (1 / 4 + p * (1 / 5 + p / 6)))))
    t = float(_NUM_SAMPLED) * l1p
    em_small = t * (1.0 + t * (1 / 2 + t * (1 / 6 + t * (1 / 24 + t / 120))))
    em = jnp.where(t > -0.125, em_small, jnp.exp(t) - 1.0)
    return -em


def _tc_body(xt_ref, g_ref, st_ref, tid_ref, sid_ref, out_ref):
    xt = xt_ref[...]
    g = g_ref[...]
    st = st_ref[...]
    tid = tid_ref[...]
    sid = sid_ref[...]

    tl = jnp.sum(xt * g, axis=0, keepdims=True)
    slt = lax.dot_general(
        st, xt, (((0,), (0,)), ((), ())),
        preferred_element_type=jnp.float32)

    tl = tl - jnp.log(_expected_count(tid.astype(jnp.float32)))
    slt = slt - jnp.log(_expected_count(sid.astype(jnp.float32)))

    slt = jnp.where(sid == tid, slt - 1e9, slt)
    srow = lax.broadcasted_iota(jnp.int32, slt.shape, 0)
    slt = jnp.where(srow < _NUM_SAMPLED, slt, -1e30)

    m = jnp.maximum(jnp.max(slt, axis=0, keepdims=True), tl)
    s = jnp.sum(jnp.exp(slt - m), axis=0, keepdims=True) + jnp.exp(tl - m)
    out_ref[...] = (jnp.log(s) + m - tl).T


def _tc_dense(xt, g, st, tid, sid):
    nblk = 4
    bb = _BATCH // nblk
    return pl.pallas_call(
        _tc_body,
        grid=(nblk,),
        in_specs=[
            pl.BlockSpec((_EMBED_DIM, bb), lambda i: (0, i)),
            pl.BlockSpec((_EMBED_DIM, bb), lambda i: (0, i)),
            pl.BlockSpec((_EMBED_DIM, _SPAD), lambda i: (0, 0)),
            pl.BlockSpec((1, bb), lambda i: (0, i)),
            pl.BlockSpec((_SPAD, 1), lambda i: (0, 0)),
        ],
        out_specs=pl.BlockSpec((bb, 1), lambda i: (i, 0)),
        out_shape=jax.ShapeDtypeStruct((_BATCH, 1), jnp.float32),
    )(xt, g, st, tid, sid)


def kernel(inputs, label_idx, embeddings, zero_bias, sampled_ids):
    del zero_bias
    tid = label_idx.reshape(-1).astype(jnp.int32)
    sid_pad = jnp.concatenate(
        [sampled_ids.astype(jnp.int32),
         jnp.zeros((_SPAD - _NUM_SAMPLED,), jnp.int32)])
    g, samp_t = _make_sc_gather()(embeddings.T, tid, sid_pad)
    loss = _tc_dense(inputs.T, g, samp_t,
                     tid.reshape(1, _BATCH), sid_pad.reshape(_SPAD, 1))
    return loss


# speedup vs baseline: 2.2924x; 1.0282x over previous
import functools

import jax
import jax.numpy as jnp
import numpy as np
from jax import lax
from jax.experimental import pallas as pl
from jax.experimental.pallas import tpu as pltpu
from jax.experimental.pallas import tpu_sc as plsc

_VOCAB = 100000
_EMBED_DIM = 64
_NUM_SAMPLED = 100
_BATCH = 4096
_SPAD = 128

_NC = 2
_NS = 16
_NW = _NC * _NS
_DPW = _EMBED_DIM // _NW


@functools.cache
def _make_sc_gather():
    @functools.partial(
        pl.kernel,
        out_type=[
            jax.ShapeDtypeStruct((_EMBED_DIM, _BATCH), jnp.float32),
            jax.ShapeDtypeStruct((_EMBED_DIM, _SPAD), jnp.float32),
        ],
        mesh=plsc.VectorSubcoreMesh(
            core_axis_name="c", subcore_axis_name="s",
            num_cores=_NC, num_subcores=_NS,
        ),
        scratch_types=[
            pltpu.VMEM((1, _VOCAB), jnp.float32),
            pltpu.VMEM((_BATCH,), jnp.int32),
            pltpu.VMEM((1, _BATCH), jnp.float32),
            pltpu.VMEM((_SPAD,), jnp.int32),
            pltpu.VMEM((1, _SPAD), jnp.float32),
        ],
        compiler_params=pltpu.CompilerParams(needs_layout_passes=False),
    )
    def _sc_gather(ett_hbm, tid_hbm, sid_hbm, g_out, samp_out,
                   row_v, tid_v, g_v, sid_v, sg_v):
        wid = lax.axis_index("s") * _NC + lax.axis_index("c")
        pltpu.sync_copy(tid_hbm, tid_v)
        pltpu.sync_copy(sid_hbm, sid_v)
        for dd in range(_DPW):
            d = wid + dd * _NW
            pltpu.sync_copy(ett_hbm.at[pl.ds(d, 1)], row_v)
            row = row_v.at[0]
            for c in range(_BATCH // 16):
                idx = tid_v[pl.ds(c * 16, 16)]
                g_v[0, pl.ds(c * 16, 16)] = plsc.load_gather(row, [idx])
            pltpu.sync_copy(g_v, g_out.at[pl.ds(d, 1)])
            for c in range(_SPAD // 16):
                idx = sid_v[pl.ds(c * 16, 16)]
                sg_v[0, pl.ds(c * 16, 16)] = plsc.load_gather(row, [idx])
            pltpu.sync_copy(sg_v, samp_out.at[pl.ds(d, 1)])

    return _sc_gather


def _expected_count(ids_f32):
    p = (jnp.log(ids_f32 + 2.0) - jnp.log(ids_f32 + 1.0)) / np.log(_VOCAB + 1.0)
    l1p = -p * (1.0 + p * (1 / 2 + p * (1 / 3 + p * (1 / 4 + p * (1 / 5 + p / 6)))))
    t = float(_NUM_SAMPLED) * l1p
    em_small = t * (1.0 + t * (1 / 2 + t * (1 / 6 + t * (1 / 24 + t / 120))))
    em = jnp.where(t > -0.125, em_small, jnp.exp(t) - 1.0)
    return -em


def _tc_body(xt_ref, g_ref, st_ref, tid_ref, sid_ref, out_ref):
    xt = xt_ref[...]
    g = g_ref[...]
    st = st_ref[...]
    tid = tid_ref[...]
    sid = sid_ref[...]

    tl = jnp.sum(xt * g, axis=0, keepdims=True)
    slt = lax.dot_general(
        st, xt, (((0,), (0,)), ((), ())),
        preferred_element_type=jnp.float32)

    tl = tl - jnp.log(_expected_count(tid.astype(jnp.float32)))
    slt = slt - jnp.log(_expected_count(sid.astype(jnp.float32)))

    slt = jnp.where(sid == tid, slt - 1e9, slt)
    srow = lax.broadcasted_iota(jnp.int32, slt.shape, 0)
    slt = jnp.where(srow < _NUM_SAMPLED, slt, -1e30)

    m = jnp.maximum(jnp.max(slt, axis=0, keepdims=True), tl)
    s = jnp.sum(jnp.exp(slt - m), axis=0, keepdims=True) + jnp.exp(tl - m)
    out_ref[...] = (jnp.log(s) + m - tl).T


def _tc_dense(xt, g, st, tid, sid):
    nblk = 2
    bb = _BATCH // nblk
    return pl.pallas_call(
        _tc_body,
        grid=(nblk,),
        in_specs=[
            pl.BlockSpec((_EMBED_DIM, bb), lambda i: (0, i)),
            pl.BlockSpec((_EMBED_DIM, bb), lambda i: (0, i)),
            pl.BlockSpec((_EMBED_DIM, _SPAD), lambda i: (0, 0)),
            pl.BlockSpec((1, bb), lambda i: (0, i)),
            pl.BlockSpec((_SPAD, 1), lambda i: (0, 0)),
        ],
        out_specs=pl.BlockSpec((bb, 1), lambda i: (i, 0)),
        out_shape=jax.ShapeDtypeStruct((_BATCH, 1), jnp.float32),
    )(xt, g, st, tid, sid)


def kernel(inputs, label_idx, embeddings, zero_bias, sampled_ids):
    del zero_bias
    tid = label_idx.reshape(-1).astype(jnp.int32)
    sid_pad = jnp.concatenate(
        [sampled_ids.astype(jnp.int32),
         jnp.zeros((_SPAD - _NUM_SAMPLED,), jnp.int32)])
    g, samp_t = _make_sc_gather()(embeddings.T, tid, sid_pad)
    loss = _tc_dense(inputs.T, g, samp_t,
                     tid.reshape(1, _BATCH), sid_pad.reshape(_SPAD, 1))
    return loss
